# baseline (speedup 1.0000x reference)
import jax
import jax.numpy as jnp
from jax.experimental import pallas as pl


def _prep_body(la_ref, adj_ref, x_ref, w0_ref, ab_ref, zb_ref, disr_ref):
    a = adj_ref[...]
    la = la_ref[...]
    deg = jnp.sum(a, axis=1, keepdims=True) + la
    dis = jnp.where(deg > 0, jax.lax.rsqrt(deg), 0.0)
    ab_ref[...] = (a * dis).astype(jnp.bfloat16)
    y = jnp.dot(x_ref[...], w0_ref[...], preferred_element_type=jnp.float32)
    z = dis * y
    zb_ref[...] = z.astype(jnp.bfloat16)
    disr_ref[...] = jnp.broadcast_to(dis, z.shape)


def _mm1_body(la_ref, ab_ref, zb_ref, zblk_ref, disr_ref, wdec_ref, mlpw_ref,
              b0_ref, mlpb_ref, h_ref, ln_ref, zwb_ref, diag2_ref):
    la = la_ref[...]
    dis = disr_ref[...]
    zblk = zblk_ref[...].astype(jnp.float32)
    acc = jnp.dot(ab_ref[...], zb_ref[...],
                  preferred_element_type=jnp.float32)
    emb = jnp.maximum(acc + la * dis * zblk + b0_ref[...], 0.0)
    h_ref[...] = emb
    ln_ref[...] = jnp.dot(emb, mlpw_ref[...],
                          preferred_element_type=jnp.float32) + mlpb_ref[...]
    w = jnp.dot(emb, wdec_ref[...], preferred_element_type=jnp.float32)
    zw = dis[:, : w.shape[1]] * w
    zwb_ref[...] = zw.astype(jnp.bfloat16)
    diag2_ref[...] = la * dis[:, : w.shape[1]] * zw


def _mm2_body(ab_ref, zwb_ref, diag2_ref, bdec_ref, out_ref):
    acc = jnp.dot(ab_ref[...], zwb_ref[...], preferred_element_type=jnp.float32)
    out_ref[...] = acc + diag2_ref[...] + bdec_ref[...]


def kernel(x, adj, loop_att, W0, b0, W_dec, b_dec, mlp_W, mlp_b):
    N, D = x.shape[1], x.shape[2]
    C = W_dec.shape[1]
    BM = 512
    NB = N // BM

    A = adj.reshape(N, N)
    x2 = x.reshape(N, D)
    la = loop_att.reshape(1, 1)
    b0r = b0.reshape(1, D)
    bdecr = b_dec.reshape(1, C)
    mlpbr = mlp_b.reshape(1, C)

    ab, zb, disr = pl.pallas_call(
        _prep_body,
        grid=(NB,),
        in_specs=[
            pl.BlockSpec((1, 1), lambda i: (0, 0)),
            pl.BlockSpec((BM, N), lambda i: (i, 0)),
            pl.BlockSpec((BM, D), lambda i: (i, 0)),
            pl.BlockSpec((D, D), lambda i: (0, 0)),
        ],
        out_specs=[
            pl.BlockSpec((BM, N), lambda i: (i, 0)),
            pl.BlockSpec((BM, D), lambda i: (i, 0)),
            pl.BlockSpec((BM, D), lambda i: (i, 0)),
        ],
        out_shape=[
            jax.ShapeDtypeStruct((N, N), jnp.bfloat16),
            jax.ShapeDtypeStruct((N, D), jnp.bfloat16),
            jax.ShapeDtypeStruct((N, D), jnp.float32),
        ],
    )(la, A, x2, W0)

    h, ln, zwb, diag2 = pl.pallas_call(
        _mm1_body,
        grid=(NB,),
        in_specs=[
            pl.BlockSpec((1, 1), lambda i: (0, 0)),
            pl.BlockSpec((BM, N), lambda i: (i, 0)),
            pl.BlockSpec((N, D), lambda i: (0, 0)),
            pl.BlockSpec((BM, D), lambda i: (i, 0)),
            pl.BlockSpec((BM, D), lambda i: (i, 0)),
            pl.BlockSpec((D, C), lambda i: (0, 0)),
            pl.BlockSpec((D, C), lambda i: (0, 0)),
            pl.BlockSpec((1, D), lambda i: (0, 0)),
            pl.BlockSpec((1, C), lambda i: (0, 0)),
        ],
        out_specs=[
            pl.BlockSpec((BM, D), lambda i: (i, 0)),
            pl.BlockSpec((BM, C), lambda i: (i, 0)),
            pl.BlockSpec((BM, C), lambda i: (i, 0)),
            pl.BlockSpec((BM, C), lambda i: (i, 0)),
        ],
        out_shape=[
            jax.ShapeDtypeStruct((N, D), jnp.float32),
            jax.ShapeDtypeStruct((N, C), jnp.float32),
            jax.ShapeDtypeStruct((N, C), jnp.bfloat16),
            jax.ShapeDtypeStruct((N, C), jnp.float32),
        ],
    )(la, ab, zb, zb, disr, W_dec, mlp_W, b0r, mlpbr)

    logits = pl.pallas_call(
        _mm2_body,
        grid=(NB,),
        in_specs=[
            pl.BlockSpec((BM, N), lambda i: (i, 0)),
            pl.BlockSpec((N, C), lambda i: (0, 0)),
            pl.BlockSpec((BM, C), lambda i: (i, 0)),
            pl.BlockSpec((1, C), lambda i: (0, 0)),
        ],
        out_specs=pl.BlockSpec((BM, C), lambda i: (i, 0)),
        out_shape=jax.ShapeDtypeStruct((N, C), jnp.float32),
    )(ab, zwb, diag2, bdecr)

    return (logits, ln[None], h)


# speedup vs baseline: 1.2744x; 1.2744x over previous
import jax
import jax.numpy as jnp
from jax.experimental import pallas as pl
from jax.experimental.pallas import tpu as pltpu

_BM = 256


def _body(la_ref, adj_ref, x_ref, w0_ref, wdec_ref, mlpw_ref, b0_ref,
          mlpb_ref, bdec_ref, h_ref, ln_ref, logits_ref,
          ab_s, zb_s, disr_s, zwb_s, diag2_s):
    p = pl.program_id(0)
    ii = pl.program_id(1)
    r0 = pl.multiple_of(ii * _BM, _BM)
    la = la_ref[...]

    @pl.when(p == 0)
    def _phase0():
        a = adj_ref[...]
        deg = jnp.sum(a, axis=1, keepdims=True) + la
        dis = jnp.where(deg > 0, jax.lax.rsqrt(deg), 0.0)
        ab_s[pl.ds(r0, _BM), :] = (a * dis).astype(jnp.bfloat16)
        y = jnp.dot(x_ref[...], w0_ref[...], preferred_element_type=jnp.float32)
        z = dis * y
        zb_s[pl.ds(r0, _BM), :] = z.astype(jnp.bfloat16)
        disr_s[pl.ds(r0, _BM), :] = jnp.broadcast_to(dis, z.shape)

    @pl.when(p == 1)
    def _phase1():
        ab = ab_s[pl.ds(r0, _BM), :]
        acc = jnp.dot(ab, zb_s[...], preferred_element_type=jnp.float32)
        dis = disr_s[pl.ds(r0, _BM), :]
        zblk = zb_s[pl.ds(r0, _BM), :].astype(jnp.float32)
        emb = jnp.maximum(acc + la * dis * zblk + b0_ref[...], 0.0)
        h_ref[...] = emb
        ln_ref[...] = jnp.dot(emb, mlpw_ref[...],
                              preferred_element_type=jnp.float32) + mlpb_ref[...]
        w = jnp.dot(emb, wdec_ref[...], preferred_element_type=jnp.float32)
        disc = dis[:, : w.shape[1]]
        zw = disc * w
        zwb_s[pl.ds(r0, _BM), :] = zw.astype(jnp.bfloat16)
        diag2_s[pl.ds(r0, _BM), :] = la * disc * zw

    @pl.when(p == 2)
    def _phase2():
        ab = ab_s[pl.ds(r0, _BM), :]
        acc = jnp.dot(ab, zwb_s[...], preferred_element_type=jnp.float32)
        logits_ref[...] = acc + diag2_s[pl.ds(r0, _BM), :] + bdec_ref[...]


def kernel(x, adj, loop_att, W0, b0, W_dec, b_dec, mlp_W, mlp_b):
    N, D = x.shape[1], x.shape[2]
    C = W_dec.shape[1]
    NB = N // _BM

    A = adj.reshape(N, N)
    x2 = x.reshape(N, D)
    la = loop_att.reshape(1, 1)
    b0r = b0.reshape(1, D)
    bdecr = b_dec.reshape(1, C)
    mlpbr = mlp_b.reshape(1, C)

    h, ln, logits = pl.pallas_call(
        _body,
        grid=(3, NB),
        in_specs=[
            pl.BlockSpec((1, 1), lambda p, i: (0, 0)),
            pl.BlockSpec((_BM, N), lambda p, i: (jnp.where(p == 0, i, NB - 1), 0)),
            pl.BlockSpec((_BM, D), lambda p, i: (jnp.where(p == 0, i, NB - 1), 0)),
            pl.BlockSpec((D, D), lambda p, i: (0, 0)),
            pl.BlockSpec((D, C), lambda p, i: (0, 0)),
            pl.BlockSpec((D, C), lambda p, i: (0, 0)),
            pl.BlockSpec((1, D), lambda p, i: (0, 0)),
            pl.BlockSpec((1, C), lambda p, i: (0, 0)),
            pl.BlockSpec((1, C), lambda p, i: (0, 0)),
        ],
        out_specs=[
            pl.BlockSpec((_BM, D),
                         lambda p, i: (jnp.where(p == 1, i, jnp.where(p == 0, 0, NB - 1)), 0)),
            pl.BlockSpec((_BM, C),
                         lambda p, i: (jnp.where(p == 1, i, jnp.where(p == 0, 0, NB - 1)), 0)),
            pl.BlockSpec((_BM, C), lambda p, i: (jnp.where(p == 2, i, 0), 0)),
        ],
        out_shape=[
            jax.ShapeDtypeStruct((N, D), jnp.float32),
            jax.ShapeDtypeStruct((N, C), jnp.float32),
            jax.ShapeDtypeStruct((N, C), jnp.float32),
        ],
        scratch_shapes=[
            pltpu.VMEM((N, N), jnp.bfloat16),
            pltpu.VMEM((N, D), jnp.bfloat16),
            pltpu.VMEM((N, D), jnp.float32),
            pltpu.VMEM((N, C), jnp.bfloat16),
            pltpu.VMEM((N, C), jnp.float32),
        ],
    )(la, A, x2, W0, W_dec, mlp_W, b0r, mlpbr, bdecr)

    return (logits, ln[None], h)
